# baseline (speedup 1.0000x reference)
import functools

import jax
import jax.numpy as jnp
from jax import lax
from jax.experimental import pallas as pl
from jax.experimental.pallas import tpu as pltpu
from jax.experimental.pallas import tpu_sc as plsc

L = 16


def _sc_body(B, T, D, CH, x_hbm, pe_hbm, table_hbm, mid_hbm, out_hbm,
             idx_v, me_v, pe_v, x_v, sem):
    c = lax.axis_index("c")
    s = lax.axis_index("s")
    nc = lax.axis_size("c")
    ns = lax.axis_size("s")
    nw = nc * ns
    wid = s * nc + c

    pltpu.sync_copy(mid_hbm, idx_v)
    pltpu.async_copy(table_hbm.at[idx_v], me_v, sem).wait()

    t_per_w = T // nw
    n_ch = t_per_w // CH
    base = wid * t_per_w

    def chunk(k, carry):
        t0 = base + k * CH
        pltpu.sync_copy(pe_hbm.at[pl.ds(t0, CH)], pe_v)
        for b in range(B):
            pltpu.sync_copy(x_hbm.at[b, pl.ds(t0, CH), :], x_v.at[b])

        def jloop(j, carry2):
            sl = pl.ds(j * L, L)
            mv = me_v[0, sl]
            for r in range(CH):
                pv = pe_v[r, sl] + mv
                for b in range(B):
                    x_v[b, r, sl] = x_v[b, r, sl] + pv
            return carry2

        lax.fori_loop(0, D // L, jloop, 0)
        for b in range(B):
            pltpu.sync_copy(x_v.at[b], out_hbm.at[b, pl.ds(t0, CH), :])
        return carry

    lax.fori_loop(0, n_ch, chunk, 0)


@jax.jit
def kernel(x, temporal_pe, modality_table, modality_id):
    B, T, D = x.shape
    CH = 8
    pe2 = temporal_pe.reshape(temporal_pe.shape[1], D)
    mid = jnp.asarray(modality_id, jnp.int32).reshape(1)

    mesh = plsc.VectorSubcoreMesh(core_axis_name="c", subcore_axis_name="s")
    body = functools.partial(_sc_body, B, T, D, CH)
    k = pl.kernel(
        body,
        mesh=mesh,
        out_type=jax.ShapeDtypeStruct((B, T, D), x.dtype),
        scratch_types=[
            pltpu.VMEM((1,), jnp.int32),
            pltpu.VMEM((1, D), jnp.float32),
            pltpu.VMEM((CH, D), jnp.float32),
            pltpu.VMEM((B, CH, D), jnp.float32),
            pltpu.SemaphoreType.DMA,
        ],
    )
    return k(x, pe2, modality_table, mid)
